# Initial kernel scaffold; baseline (speedup 1.0000x reference)
import jax
import jax.numpy as jnp
from jax.experimental import pallas as pl


def kernel(x, embed, pos_embed):
    raise NotImplementedError("write your pallas kernel here")


# speedup vs baseline: 4.3689x; 4.3689x over previous
import functools

import jax
import jax.numpy as jnp
from jax import lax
from jax.experimental import pallas as pl
from jax.experimental.pallas import tpu as pltpu
from jax.experimental.pallas import tpu_sc as plsc

NC = 2
NS = 16
NW = NC * NS
HALF = 100


def _sc_embed(x_r, embed, pos_embed):
    n_half, _ = x_r.shape
    V, D = embed.shape
    L, _ = pos_embed.shape
    BL = n_half * HALF
    B = BL // L
    BPW = B // NW

    mesh = plsc.VectorSubcoreMesh(core_axis_name="c", subcore_axis_name="s")

    @functools.partial(
        pl.kernel,
        mesh=mesh,
        out_type=jax.ShapeDtypeStruct((BL, D), jnp.float32),
        scratch_types=[
            pltpu.VMEM((L, D), jnp.float32),
            pltpu.VMEM((2, HALF), jnp.int32),
            pltpu.VMEM((L, D), jnp.float32),
            pltpu.SemaphoreType.DMA,
            pltpu.SemaphoreType.DMA,
        ],
    )
    def k(x_hbm, embed_hbm, pos_hbm, out_hbm, pos_v, idx_v, rows_v, sem0, sem1):
        wid = lax.axis_index("s") * NC + lax.axis_index("c")
        b0 = wid * BPW
        pltpu.sync_copy(pos_hbm, pos_v)

        def per_b(i, carry):
            gb = b0 + i
            pltpu.sync_copy(x_hbm.at[pl.ds(gb * 2, 2)], idx_v)
            cp0 = pltpu.make_async_copy(
                embed_hbm.at[idx_v.at[0]], rows_v.at[pl.ds(0, HALF)], sem0)
            cp1 = pltpu.make_async_copy(
                embed_hbm.at[idx_v.at[1]], rows_v.at[pl.ds(HALF, HALF)], sem1)
            cp0.start()
            cp1.start()
            cp0.wait()
            cp1.wait()

            def add_l(l, c):
                for j in range(D // 16):
                    sl = pl.ds(j * 16, 16)
                    rows_v[l, sl] = rows_v[l, sl] + pos_v[l, sl]
                return c

            lax.fori_loop(0, L, add_l, 0)
            pltpu.sync_copy(rows_v, out_hbm.at[pl.ds(gb * L, L)])
            return carry

        lax.fori_loop(0, BPW, per_b, 0)

    return k(x_r, embed, pos_embed)


def kernel(x, embed, pos_embed):
    B, L = x.shape
    V, D = embed.shape
    x_r = x.astype(jnp.int32).reshape(B * L // HALF, HALF)
    out = _sc_embed(x_r, embed, pos_embed)
    return out.reshape(B, L, D)


# speedup vs baseline: 7.6679x; 1.7551x over previous
import functools

import jax
import jax.numpy as jnp
from jax import lax
from jax.experimental import pallas as pl
from jax.experimental.pallas import tpu as pltpu
from jax.experimental.pallas import tpu_sc as plsc

NC = 2
NS = 16
NW = NC * NS
HALF = 100


def _sc_embed(x_r, embed, pos_embed):
    n_half, _ = x_r.shape
    V, D = embed.shape
    L, _ = pos_embed.shape
    BL = n_half * HALF
    B = BL // L
    BPW = B // NW
    assert BPW % 2 == 0

    mesh = plsc.VectorSubcoreMesh(core_axis_name="c", subcore_axis_name="s")

    @functools.partial(
        pl.kernel,
        mesh=mesh,
        out_type=jax.ShapeDtypeStruct((BL, D), jnp.float32),
        scratch_types=[
            pltpu.VMEM((L, D), jnp.float32),
            pltpu.VMEM((2, 2, HALF), jnp.int32),
            pltpu.VMEM((2, L, D), jnp.float32),
            pltpu.SemaphoreType.DMA,
            pltpu.SemaphoreType.DMA,
            pltpu.SemaphoreType.DMA,
            pltpu.SemaphoreType.DMA,
            pltpu.SemaphoreType.DMA,
            pltpu.SemaphoreType.DMA,
            pltpu.SemaphoreType.DMA,
        ],
    )
    def k(x_hbm, embed_hbm, pos_hbm, out_hbm, pos_v, idx_v, rows_v,
          si0, si1, sg0, sg1, so0, so1, sp):
        sem_i = (si0, si1)
        sem_g = (sg0, sg1)
        sem_o = (so0, so1)
        wid = lax.axis_index("s") * NC + lax.axis_index("c")
        b0 = wid * BPW

        def idx_copy(b, buf):
            return pltpu.make_async_copy(
                x_hbm.at[pl.ds(b * 2, 2)], idx_v.at[buf], sem_i[buf])

        def gather_copy(h, buf):
            return pltpu.make_async_copy(
                embed_hbm.at[idx_v.at[buf, h]],
                rows_v.at[buf, pl.ds(h * HALF, HALF)], sem_g[buf])

        def out_copy(b, buf):
            return pltpu.make_async_copy(
                rows_v.at[buf], out_hbm.at[pl.ds(b * L, L)], sem_o[buf])

        pos_cp = pltpu.make_async_copy(pos_hbm, pos_v, sp)
        pos_cp.start()
        idx_copy(b0, 0).start()
        idx_copy(b0 + 1, 1).start()
        idx_copy(b0, 0).wait()
        gather_copy(0, 0).start()
        gather_copy(1, 0).start()
        pos_cp.wait()

        @pl.loop(0, BPW, step=2)
        def per_pair(i):
            for cur in range(2):
                nxt = 1 - cur
                ii = b0 + i + cur
                gather_copy(0, cur).wait()
                gather_copy(1, cur).wait()

                @pl.when(i + cur + 1 < BPW)
                def _():
                    idx_copy(ii + 1, nxt).wait()

                    @pl.when(i + cur >= 1)
                    def _():
                        out_copy(ii - 1, nxt).wait()

                    gather_copy(0, nxt).start()
                    gather_copy(1, nxt).start()

                @pl.when(i + cur + 2 < BPW)
                def _():
                    idx_copy(ii + 2, cur).start()

                def add_l(l, c):
                    for j in range(D // 16):
                        sl = pl.ds(j * 16, 16)
                        rows_v[cur, l, sl] = rows_v[cur, l, sl] + pos_v[l, sl]
                    return c

                lax.fori_loop(0, L, add_l, 0)
                out_copy(ii, cur).start()

        out_copy(b0 + BPW - 2, 0).wait()
        out_copy(b0 + BPW - 1, 1).wait()

    return k(x_r, embed, pos_embed)


def kernel(x, embed, pos_embed):
    B, L = x.shape
    V, D = embed.shape
    x_r = x.astype(jnp.int32).reshape(B * L // HALF, HALF)
    out = _sc_embed(x_r, embed, pos_embed)
    return out.reshape(B, L, D)


# speedup vs baseline: 9.0955x; 1.1862x over previous
import functools

import jax
import jax.numpy as jnp
from jax import lax
from jax.experimental import pallas as pl
from jax.experimental.pallas import tpu as pltpu
from jax.experimental.pallas import tpu_sc as plsc

NC = 2
NS = 16
NW = NC * NS
HALF = 100
SPLIT = 104
NBUF = 4


def _sc_embed(x_r, embed, pos_embed):
    n_half, _ = x_r.shape
    V, D = embed.shape
    L, _ = pos_embed.shape
    BL = n_half * HALF
    B = BL // L
    BPW = B // NW
    assert BPW % NBUF == 0 and L == 2 * HALF

    mesh = plsc.VectorSubcoreMesh(core_axis_name="c", subcore_axis_name="s")

    @functools.partial(
        pl.kernel,
        mesh=mesh,
        out_type=jax.ShapeDtypeStruct((BL, D), jnp.float32),
        scratch_types=[
            pltpu.VMEM((L, D), jnp.float32),
            pltpu.VMEM((NBUF, 2, HALF), jnp.int32),
            pltpu.VMEM((NBUF, L, D), jnp.float32),
            pltpu.SemaphoreType.DMA,
            pltpu.SemaphoreType.DMA,
            pltpu.SemaphoreType.DMA,
            pltpu.SemaphoreType.DMA,
            pltpu.SemaphoreType.DMA,
            pltpu.SemaphoreType.DMA,
            pltpu.SemaphoreType.DMA,
            pltpu.SemaphoreType.DMA,
            pltpu.SemaphoreType.DMA,
            pltpu.SemaphoreType.DMA,
            pltpu.SemaphoreType.DMA,
            pltpu.SemaphoreType.DMA,
            pltpu.SemaphoreType.DMA,
        ],
    )
    def k(x_hbm, embed_hbm, pos_hbm, out_hbm, pos_v, idx_v, rows_v,
          si0, si1, si2, si3, sg0, sg1, sg2, sg3, so0, so1, so2, so3, sp):
        sem_i = (si0, si1, si2, si3)
        sem_g = (sg0, sg1, sg2, sg3)
        sem_o = (so0, so1, so2, so3)
        wid = lax.axis_index("s") * NC + lax.axis_index("c")
        b0 = wid * BPW

        def idx_copy(b, buf):
            return pltpu.make_async_copy(
                x_hbm.at[pl.ds((b0 + b) * 2, 2)], idx_v.at[buf], sem_i[buf])

        def gather_copy(h, buf):
            return pltpu.make_async_copy(
                embed_hbm.at[idx_v.at[buf, h]],
                rows_v.at[buf, pl.ds(h * HALF, HALF)], sem_g[buf])

        def out_copy(b, buf, part):
            lo, sz = (0, SPLIT) if part == 0 else (SPLIT, L - SPLIT)
            return pltpu.make_async_copy(
                rows_v.at[buf, pl.ds(lo, sz)],
                out_hbm.at[pl.ds((b0 + b) * L + lo, sz)], sem_o[buf])

        def start_gather(b, buf):
            gather_copy(0, buf).start()
            gather_copy(1, buf).start()

        def wait_gather(buf):
            gather_copy(0, buf).wait()
            gather_copy(1, buf).wait()

        def wait_out(b, buf):
            out_copy(b, buf, 0).wait()
            out_copy(b, buf, 1).wait()

        pos_cp = pltpu.make_async_copy(pos_hbm, pos_v, sp)
        pos_cp.start()
        for b in range(NBUF):
            idx_copy(b, b).start()
        idx_copy(0, 0).wait()
        start_gather(0, 0)
        idx_copy(1, 1).wait()
        start_gather(1, 1)
        pos_cp.wait()

        @pl.loop(0, BPW, step=NBUF)
        def per_ring(i):
            for cur in range(NBUF):
                ii = i + cur
                nb = (cur + 2) % NBUF
                wait_gather(cur)

                @pl.when(ii + NBUF < BPW)
                def _():
                    idx_copy(ii + NBUF, cur).start()

                @pl.when(ii + 2 < BPW)
                def _():
                    idx_copy(ii + 2, nb).wait()

                    @pl.when(ii >= 2)
                    def _():
                        wait_out(ii - 2, nb)

                    start_gather(ii + 2, nb)

                def add_l(lo, hi):
                    def body(l, c):
                        for j in range(D // 16):
                            sl = pl.ds(j * 16, 16)
                            rows_v[cur, l, sl] = (
                                rows_v[cur, l, sl] + pos_v[l, sl])
                        return c
                    lax.fori_loop(lo, hi, body, 0)

                add_l(0, SPLIT)
                out_copy(ii, cur, 0).start()
                add_l(SPLIT, L)
                out_copy(ii, cur, 1).start()

        for t in range(NBUF):
            b = BPW - NBUF + t
            wait_out(b, b % NBUF)

    return k(x_r, embed, pos_embed)


def kernel(x, embed, pos_embed):
    B, L = x.shape
    V, D = embed.shape
    x_r = x.astype(jnp.int32).reshape(B * L // HALF, HALF)
    out = _sc_embed(x_r, embed, pos_embed)
    return out.reshape(B, L, D)


# speedup vs baseline: 9.1285x; 1.0036x over previous
import functools

import jax
import jax.numpy as jnp
from jax import lax
from jax.experimental import pallas as pl
from jax.experimental.pallas import tpu as pltpu
from jax.experimental.pallas import tpu_sc as plsc

NC = 2
NS = 16
NW = NC * NS
HALF = 100
SPLIT = 104
NBUF = 4


def _sc_embed(x_r, embed, pos_embed):
    n_half, _ = x_r.shape
    V, D = embed.shape
    L, _ = pos_embed.shape
    BL = n_half * HALF
    B = BL // L
    BPW = B // NW
    assert BPW % NBUF == 0 and L == 2 * HALF

    mesh = plsc.VectorSubcoreMesh(core_axis_name="c", subcore_axis_name="s")

    @functools.partial(
        pl.kernel,
        mesh=mesh,
        out_type=jax.ShapeDtypeStruct((BL, D), jnp.float32),
        scratch_types=[
            pltpu.VMEM((L, D), jnp.float32),
            pltpu.VMEM((NBUF, 2, HALF), jnp.int32),
            pltpu.VMEM((NBUF, L, D), jnp.float32),
            pltpu.SemaphoreType.DMA,
            pltpu.SemaphoreType.DMA,
            pltpu.SemaphoreType.DMA,
            pltpu.SemaphoreType.DMA,
            pltpu.SemaphoreType.DMA,
            pltpu.SemaphoreType.DMA,
            pltpu.SemaphoreType.DMA,
            pltpu.SemaphoreType.DMA,
            pltpu.SemaphoreType.DMA,
            pltpu.SemaphoreType.DMA,
            pltpu.SemaphoreType.DMA,
            pltpu.SemaphoreType.DMA,
            pltpu.SemaphoreType.DMA,
        ],
    )
    def k(x_hbm, embed_hbm, pos_hbm, out_hbm, pos_v, idx_v, rows_v,
          si0, si1, si2, si3, sg0, sg1, sg2, sg3, so0, so1, so2, so3, sp):
        sem_i = (si0, si1, si2, si3)
        sem_g = (sg0, sg1, sg2, sg3)
        sem_o = (so0, so1, so2, so3)
        wid = lax.axis_index("s") * NC + lax.axis_index("c")
        b0 = wid * BPW

        def idx_copy(b, buf):
            return pltpu.make_async_copy(
                x_hbm.at[pl.ds((b0 + b) * 2, 2)], idx_v.at[buf], sem_i[buf])

        def gather_copy(h, buf):
            return pltpu.make_async_copy(
                embed_hbm.at[idx_v.at[buf, h]],
                rows_v.at[buf, pl.ds(h * HALF, HALF)], sem_g[buf])

        def out_copy(b, buf, part):
            lo, sz = (0, SPLIT) if part == 0 else (SPLIT, L - SPLIT)
            return pltpu.make_async_copy(
                rows_v.at[buf, pl.ds(lo, sz)],
                out_hbm.at[pl.ds((b0 + b) * L + lo, sz)], sem_o[buf])

        def start_gather(b, buf):
            gather_copy(0, buf).start()
            gather_copy(1, buf).start()

        def wait_gather(buf):
            gather_copy(0, buf).wait()
            gather_copy(1, buf).wait()

        def wait_out(b, buf):
            out_copy(b, buf, 0).wait()
            out_copy(b, buf, 1).wait()

        pos_cp = pltpu.make_async_copy(pos_hbm, pos_v, sp)
        pos_cp.start()
        for b in range(NBUF):
            idx_copy(b, b).start()
        idx_copy(0, 0).wait()
        start_gather(0, 0)
        idx_copy(1, 1).wait()
        start_gather(1, 1)
        pos_cp.wait()

        @pl.loop(0, BPW, step=NBUF)
        def per_ring(i):
            for cur in range(NBUF):
                ii = i + cur
                nb = (cur + 2) % NBUF

                @pl.when(ii + 2 < BPW)
                def _():
                    idx_copy(ii + 2, nb).wait()

                    @pl.when(ii >= 2)
                    def _():
                        wait_out(ii - 2, nb)

                    start_gather(ii + 2, nb)

                wait_gather(cur)

                @pl.when(ii + NBUF < BPW)
                def _():
                    idx_copy(ii + NBUF, cur).start()

                def add_l(lo, hi):
                    def body(l, c):
                        for j in range(D // 16):
                            sl = pl.ds(j * 16, 16)
                            rows_v[cur, l, sl] = (
                                rows_v[cur, l, sl] + pos_v[l, sl])
                        return c
                    lax.fori_loop(lo, hi, body, 0)

                add_l(0, SPLIT)
                out_copy(ii, cur, 0).start()
                add_l(SPLIT, L)
                out_copy(ii, cur, 1).start()

        for t in range(NBUF):
            b = BPW - NBUF + t
            wait_out(b, b % NBUF)

    return k(x_r, embed, pos_embed)


def kernel(x, embed, pos_embed):
    B, L = x.shape
    V, D = embed.shape
    x_r = x.astype(jnp.int32).reshape(B * L // HALF, HALF)
    out = _sc_embed(x_r, embed, pos_embed)
    return out.reshape(B, L, D)
